# baseline (speedup 1.0000x reference)
import functools

import jax
import jax.numpy as jnp
from jax import lax
from jax.experimental import pallas as pl
from jax.experimental.pallas import tpu as pltpu
from jax.experimental.pallas import tpu_sc as plsc

N_NODES = 10000
N_EDGES = 320000
D = 128

NC = 2
NS = 16
NW = NC * NS
CHUNK = 128
KBLK = 8
GW = 2
NBLK = 10
E_PER_T = NBLK * KBLK * CHUNK
ACC_ROWS = 10240
SROWS = ACC_ROWS // NS


def _sc_gather_scatter(src_h, src_idx, dst_idx):
    mesh = plsc.VectorSubcoreMesh(core_axis_name="c", subcore_axis_name="s")

    @functools.partial(
        pl.kernel,
        out_type=jax.ShapeDtypeStruct((NC, ACC_ROWS, D), jnp.float32),
        mesh=mesh,
        scratch_types=[
            pltpu.VMEM((KBLK, CHUNK), jnp.int32),
            pltpu.VMEM((KBLK, CHUNK), jnp.int32),
            pltpu.VMEM((KBLK, CHUNK), jnp.int32),
            pltpu.VMEM((KBLK, CHUNK), jnp.int32),
            pltpu.VMEM((CHUNK, D), jnp.float32),
            pltpu.VMEM((CHUNK, D), jnp.float32),
            pltpu.SemaphoreType.DMA,
            pltpu.SemaphoreType.DMA,
            pltpu.SemaphoreType.DMA,
            pltpu.SemaphoreType.DMA,
            pltpu.SemaphoreType.DMA,
            pltpu.SemaphoreType.DMA,
            pltpu.VMEM_SHARED((ACC_ROWS, D), jnp.float32),
        ],
    )
    def k(h_hbm, src_hbm, dst_hbm, out_hbm,
          sidx_a, sidx_b, didx_a, didx_b, rows_a, rows_b,
          sem_s, sem_d, sem_ra, sem_rb, sem_sa, sem_sb, acc):
        c = lax.axis_index("c")
        s = lax.axis_index("s")

        def zrow(i, carry):
            for j in range(D // 16):
                rows_a[i, pl.ds(j * 16, 16)] = jnp.zeros((16,), jnp.float32)
            return carry
        lax.fori_loop(0, CHUNK, zrow, 0)
        base = s * SROWS
        for t in range(SROWS // CHUNK):
            pltpu.sync_copy(rows_a, acc.at[pl.ds(base + t * CHUNK, CHUNK)])
        plsc.subcore_barrier()

        def iload(blk, sbuf, dbuf):
            pltpu.async_copy(src_hbm.at[c, s, blk], sbuf, sem_s)
            pltpu.async_copy(dst_hbm.at[c, s, blk], dbuf, sem_d)

        def iwait(sbuf, dbuf):
            pltpu.make_async_copy(src_hbm.at[c, s, 0], sbuf, sem_s).wait()
            pltpu.make_async_copy(dst_hbm.at[c, s, 0], dbuf, sem_d).wait()

        def gth(buf, idx, sem):
            pltpu.async_copy(h_hbm.at[idx], buf, sem)

        def gwt(buf, sem):
            pltpu.make_async_copy(h_hbm.at[sidx_a.at[0]], buf, sem).wait()

        def sct(buf, idx, sem):
            pltpu.async_copy(buf, acc.at[idx], sem, add=True)

        def swt(buf, sem):
            pltpu.make_async_copy(buf, acc.at[didx_a.at[0]], sem).wait()

        def run_block(sbuf, dbuf):
            gth(rows_a, sbuf.at[0], sem_ra)
            gth(rows_b, sbuf.at[1], sem_rb)

            def pair(p, carry):
                j0 = 2 * p
                gwt(rows_a, sem_ra)
                sct(rows_a, dbuf.at[j0], sem_sa)
                gwt(rows_b, sem_rb)
                sct(rows_b, dbuf.at[j0 + 1], sem_sb)
                swt(rows_a, sem_sa)
                gth(rows_a, sbuf.at[j0 + 2], sem_ra)
                swt(rows_b, sem_sb)
                gth(rows_b, sbuf.at[j0 + 3], sem_rb)
                return carry
            lax.fori_loop(0, KBLK // 2 - 1, pair, 0)

            j0 = KBLK - 2
            gwt(rows_a, sem_ra)
            sct(rows_a, dbuf.at[j0], sem_sa)
            gwt(rows_b, sem_rb)
            sct(rows_b, dbuf.at[j0 + 1], sem_sb)
            swt(rows_a, sem_sa)
            swt(rows_b, sem_sb)

        iload(0, sidx_a, didx_a)
        iwait(sidx_a, didx_a)
        iload(1, sidx_b, didx_b)

        def body(bp, carry):
            b0 = 2 * bp
            run_block(sidx_a, didx_a)
            iwait(sidx_b, didx_b)
            iload(b0 + 2, sidx_a, didx_a)
            run_block(sidx_b, didx_b)
            iwait(sidx_a, didx_a)
            iload(b0 + 3, sidx_b, didx_b)
            return carry
        lax.fori_loop(0, NBLK // 2 - 1, body, 0)

        run_block(sidx_a, didx_a)
        iwait(sidx_b, didx_b)
        run_block(sidx_b, didx_b)

        plsc.subcore_barrier()
        pltpu.sync_copy(acc.at[pl.ds(base, SROWS)],
                        out_hbm.at[c].at[pl.ds(base, SROWS)])

    return k(src_h, src_idx, dst_idx)


def _tc_linear(acc2, W, b2):
    BR = 2000

    def body(a0_ref, a1_ref, w_ref, b_ref, o_ref):
        x = a0_ref[0] + a1_ref[0]
        o_ref[...] = lax.dot_general(
            x, w_ref[...], (((1,), (1,)), ((), ())),
            preferred_element_type=jnp.float32) + b_ref[...]

    return pl.pallas_call(
        body,
        grid=(N_NODES // BR,),
        in_specs=[
            pl.BlockSpec((1, BR, D), lambda i: (0, i, 0)),
            pl.BlockSpec((1, BR, D), lambda i: (1, i, 0)),
            pl.BlockSpec((D, D), lambda i: (0, 0)),
            pl.BlockSpec((1, D), lambda i: (0, 0)),
        ],
        out_specs=pl.BlockSpec((BR, D), lambda i: (i, 0)),
        out_shape=jax.ShapeDtypeStruct((N_NODES, D), jnp.float32),
    )(acc2, acc2, W, b2)


def kernel(src_h, edge_index, W, b):
    pad = NW * E_PER_T - N_EDGES
    src = jnp.concatenate([edge_index[0], jnp.zeros((pad,), jnp.int32)])
    dst = jnp.concatenate([edge_index[1], jnp.full((pad,), N_NODES, jnp.int32)])
    src_idx = src.reshape(NC, NS, NBLK, KBLK, CHUNK)
    dst_idx = dst.reshape(NC, NS, NBLK, KBLK, CHUNK)
    acc2 = _sc_gather_scatter(src_h, src_idx, dst_idx)
    return _tc_linear(acc2, W, b.reshape(1, D))


# speedup vs baseline: 1.4703x; 1.4703x over previous
import functools

import jax
import jax.numpy as jnp
from jax import lax
from jax.experimental import pallas as pl
from jax.experimental.pallas import tpu as pltpu
from jax.experimental.pallas import tpu_sc as plsc

N_NODES = 10000
N_EDGES = 320000
D = 128

NC = 2
NS = 16
NW = NC * NS
CHUNK = 128
N_CHUNKS = 79
P_PER_W = N_CHUNKS * CHUNK
ACC_ROWS = 10240
ZROWS = ACC_ROWS // NS
OROWS = ACC_ROWS // NS


def _sc_gather_scatter(src_h, src_idx, dst_idx):
    mesh = plsc.VectorSubcoreMesh(core_axis_name="c", subcore_axis_name="s")

    @functools.partial(
        pl.kernel,
        out_type=jax.ShapeDtypeStruct((NC, ACC_ROWS, D), jnp.float32),
        mesh=mesh,
        scratch_types=[
            pltpu.VMEM((N_CHUNKS, CHUNK), jnp.int32),
            pltpu.VMEM((N_CHUNKS, CHUNK), jnp.int32),
            pltpu.VMEM((CHUNK, D), jnp.float32),
            pltpu.VMEM_SHARED((ACC_ROWS, D), jnp.float32),
            pltpu.SemaphoreType.DMA,
        ],
    )
    def k(h_hbm, src_hbm, dst_hbm, out_hbm, src_v, dst_v, rows_v, acc, sem):
        c = lax.axis_index("c")
        s = lax.axis_index("s")

        pltpu.sync_copy(src_hbm.at[c, s], src_v)
        pltpu.sync_copy(dst_hbm.at[c, s], dst_v)

        def zrow(i, carry):
            for j in range(D // 16):
                rows_v[i, pl.ds(j * 16, 16)] = jnp.zeros((16,), jnp.float32)
            return carry
        lax.fori_loop(0, CHUNK, zrow, 0)
        zbase = s * ZROWS
        for t in range(ZROWS // CHUNK):
            pltpu.sync_copy(rows_v, acc.at[pl.ds(zbase + t * CHUNK, CHUNK)])
        plsc.subcore_barrier()

        def body(j, carry):
            pltpu.async_copy(h_hbm.at[src_v.at[j]], rows_v, sem).wait()
            pltpu.sync_copy(rows_v, acc.at[dst_v.at[j]], add=True)
            return carry
        lax.fori_loop(0, N_CHUNKS, body, 0)
        plsc.subcore_barrier()

        obase = s * OROWS
        pltpu.sync_copy(acc.at[pl.ds(obase, OROWS)],
                        out_hbm.at[c].at[pl.ds(obase, OROWS)])

    return k(src_h, src_idx, dst_idx)


def _tc_linear(acc2, W, b2):
    BR = 2000

    def body(a0_ref, a1_ref, w_ref, b_ref, o_ref):
        x = a0_ref[0] + a1_ref[0]
        o_ref[...] = lax.dot_general(
            x, w_ref[...], (((1,), (1,)), ((), ())),
            preferred_element_type=jnp.float32) + b_ref[...]

    return pl.pallas_call(
        body,
        grid=(N_NODES // BR,),
        in_specs=[
            pl.BlockSpec((1, BR, D), lambda i: (0, i, 0)),
            pl.BlockSpec((1, BR, D), lambda i: (1, i, 0)),
            pl.BlockSpec((D, D), lambda i: (0, 0)),
            pl.BlockSpec((1, D), lambda i: (0, 0)),
        ],
        out_specs=pl.BlockSpec((BR, D), lambda i: (i, 0)),
        out_shape=jax.ShapeDtypeStruct((N_NODES, D), jnp.float32),
    )(acc2, acc2, W, b2)


def kernel(src_h, edge_index, W, b):
    pad = NW * P_PER_W - N_EDGES
    src = jnp.concatenate([edge_index[0], jnp.zeros((pad,), jnp.int32)])
    dst = jnp.concatenate([edge_index[1], jnp.full((pad,), N_NODES, jnp.int32)])
    src_idx = src.reshape(NC, NS, N_CHUNKS, CHUNK)
    dst_idx = dst.reshape(NC, NS, N_CHUNKS, CHUNK)
    acc2 = _sc_gather_scatter(src_h, src_idx, dst_idx)
    return _tc_linear(acc2, W, b.reshape(1, D))
